# Initial kernel scaffold; baseline (speedup 1.0000x reference)
import jax
import jax.numpy as jnp
from jax.experimental import pallas as pl


def kernel(pos, classes, edges, in_W, in_b, m1_W, m1_b, m2_W, m2_b, m3_W, m3_b, gn_Wih, gn_Whh, gn_bih, gn_bhh, ge_Wih, ge_Whh, ge_bih, ge_bhh):
    raise NotImplementedError("write your pallas kernel here")


# speedup vs baseline: 4.7770x; 4.7770x over previous
import functools

import jax
import jax.numpy as jnp
from jax import lax
from jax.experimental import pallas as pl
from jax.experimental.pallas import tpu as pltpu
from jax.experimental.pallas import tpu_sc as plsc

NC, NS = 2, 16
NW = NC * NS
D = 32
ITERS = 6


def _init_body(classes_ref, pos_ref, inw_ref, inb_ref, ws_ref, wd_ref, wp_ref,
               m1b_ref, nf_ref, as_ref, ad_ref):
    nf = jnp.dot(classes_ref[...], inw_ref[...],
                 preferred_element_type=jnp.float32) + inb_ref[...]
    posw = jnp.dot(pos_ref[...], wp_ref[...], preferred_element_type=jnp.float32)
    nf_ref[...] = nf
    as_ref[...] = jnp.dot(nf, ws_ref[...], preferred_element_type=jnp.float32) - posw
    ad_ref[...] = (jnp.dot(nf, wd_ref[...], preferred_element_type=jnp.float32)
                   + posw + m1b_ref[...])


def _mlp_body(g0_ref, g1_ref, w2_ref, b2_ref, w3_ref, b3_ref, msg_ref):
    h1 = jnp.maximum(g0_ref[...] + g1_ref[...], 0.0)
    h2 = jnp.maximum(
        jnp.dot(h1, w2_ref[...], preferred_element_type=jnp.float32) + b2_ref[...],
        0.0)
    msg_ref[...] = (jnp.dot(h2, w3_ref[...], preferred_element_type=jnp.float32)
                    + b3_ref[...])


def _gru_body(a_ref, nf_ref, pos_ref, wih_ref, whh_ref, bih_ref, bhh_ref,
              ws_ref, wd_ref, wp_ref, m1b_ref, nfo_ref, as_ref, ad_ref):
    a = a_ref[...]
    h = nf_ref[...]
    gi = jnp.dot(a, wih_ref[...], preferred_element_type=jnp.float32) + bih_ref[...]
    gh = jnp.dot(h, whh_ref[...], preferred_element_type=jnp.float32) + bhh_ref[...]
    r = jax.nn.sigmoid(gi[:, 0:D] + gh[:, 0:D])
    z = jax.nn.sigmoid(gi[:, D:2 * D] + gh[:, D:2 * D])
    n = jnp.tanh(gi[:, 2 * D:3 * D] + r * gh[:, 2 * D:3 * D])
    nf_new = (1.0 - z) * n + z * h
    posw = jnp.dot(pos_ref[...], wp_ref[...], preferred_element_type=jnp.float32)
    nfo_ref[...] = nf_new
    as_ref[...] = (jnp.dot(nf_new, ws_ref[...], preferred_element_type=jnp.float32)
                   - posw)
    ad_ref[...] = (jnp.dot(nf_new, wd_ref[...], preferred_element_type=jnp.float32)
                   + posw + m1b_ref[...])


def _whole(shape):
    return pl.BlockSpec(shape, lambda i: (0,) * len(shape))


def _gather_body(per_w, gch, as_hbm, ad_hbm, i0_hbm, i1_hbm, g0_hbm, g1_hbm,
                 idx0_v, idx1_v, r0_v, r1_v, sem):
    wid = lax.axis_index("s") * NC + lax.axis_index("c")
    base = wid * per_w

    def step(k, carry):
        off = base + k * gch
        pltpu.sync_copy(i0_hbm.at[pl.ds(off, gch)], idx0_v)
        pltpu.sync_copy(i1_hbm.at[pl.ds(off, gch)], idx1_v)
        pltpu.async_copy(as_hbm.at[idx0_v], r0_v, sem).wait()
        pltpu.async_copy(ad_hbm.at[idx1_v], r1_v, sem).wait()
        pltpu.sync_copy(r0_v, g0_hbm.at[pl.ds(off, gch)])
        pltpu.sync_copy(r1_v, g1_hbm.at[pl.ds(off, gch)])
        return carry

    lax.fori_loop(0, per_w // gch, step, 0)


def _scatter_body(half, e2, msg_hbm, sidx_hbm, z_hbm, out_hbm, acc_sh, idx_v, m_v):
    c = lax.axis_index("c")
    s = lax.axis_index("s")
    sp_stripe = (half + NS) // NS
    out_stripe = half // NS
    n_idx_rows = e2 // 100
    rows_per_s = n_idx_rows // NS
    kr = 8
    n_chunks = rows_per_s // kr

    pltpu.sync_copy(z_hbm, acc_sh.at[pl.ds(s * sp_stripe, sp_stripe)])
    plsc.subcore_barrier()

    def step(k, carry):
        irow = c * n_idx_rows + s * rows_per_s + k * kr
        mrow = (s * rows_per_s + k * kr) * 100
        pltpu.sync_copy(sidx_hbm.at[pl.ds(irow, kr)], idx_v)
        pltpu.sync_copy(msg_hbm.at[pl.ds(mrow, kr * 100)], m_v)
        for j in range(kr):
            pltpu.sync_copy(m_v.at[pl.ds(j * 100, 100)],
                            acc_sh.at[idx_v.at[j]], add=True)
        return carry

    lax.fori_loop(0, n_chunks, step, 0)
    plsc.subcore_barrier()
    pltpu.sync_copy(acc_sh.at[pl.ds(s * out_stripe, out_stripe)],
                    out_hbm.at[pl.ds(c * half + s * out_stripe, out_stripe)])


def kernel(pos, classes, edges, in_W, in_b, m1_W, m1_b, m2_W, m2_b, m3_W, m3_b,
           gn_Wih, gn_Whh, gn_bih, gn_bhh, ge_Wih, ge_Whh, ge_bih, ge_bhh):
    N = pos.shape[0]
    E = edges.shape[1]
    E2 = 2 * E
    HALF = N // 2
    C = classes.shape[1]

    es0 = jnp.concatenate([edges[0], edges[1]])
    es1 = jnp.concatenate([edges[1], edges[0]])
    i_lo = jnp.where(es0 < HALF, es0, HALF)
    i_hi = jnp.where(es0 >= HALF, es0 - HALF, HALF)
    sidx = jnp.concatenate([i_lo, i_hi]).reshape(2 * (E2 // 100), 100)
    ws = m1_W[0:D]
    wd = m1_W[D:2 * D]
    wp = m1_W[2 * D:]
    in_b2 = in_b.reshape(1, D)
    m1b2 = m1_b.reshape(1, D)
    m2b2 = m2_b.reshape(1, D)
    m3b2 = m3_b.reshape(1, D)
    bih2 = gn_bih.reshape(1, 3 * D)
    bhh2 = gn_bhh.reshape(1, 3 * D)
    sp_stripe = (HALF + NS) // NS
    zstripe = jnp.zeros((sp_stripe, D), jnp.float32)

    BN = 2000
    fdt = jax.ShapeDtypeStruct
    init_call = pl.pallas_call(
        _init_body,
        grid=(N // BN,),
        in_specs=[pl.BlockSpec((BN, C), lambda i: (i, 0)),
                  pl.BlockSpec((BN, 3), lambda i: (i, 0)),
                  _whole((C, D)), _whole((1, D)), _whole((D, D)),
                  _whole((D, D)), _whole((3, D)), _whole((1, D))],
        out_specs=[pl.BlockSpec((BN, D), lambda i: (i, 0))] * 3,
        out_shape=[fdt((N, D), jnp.float32)] * 3,
    )

    BE = 8000
    mlp_call = pl.pallas_call(
        _mlp_body,
        grid=(E2 // BE,),
        in_specs=[pl.BlockSpec((BE, D), lambda i: (i, 0)),
                  pl.BlockSpec((BE, D), lambda i: (i, 0)),
                  _whole((D, D)), _whole((1, D)), _whole((D, D)), _whole((1, D))],
        out_specs=pl.BlockSpec((BE, D), lambda i: (i, 0)),
        out_shape=fdt((E2, D), jnp.float32),
    )

    gru_call = pl.pallas_call(
        _gru_body,
        grid=(N // BN,),
        in_specs=[pl.BlockSpec((BN, D), lambda i: (i, 0)),
                  pl.BlockSpec((BN, D), lambda i: (i, 0)),
                  pl.BlockSpec((BN, 3), lambda i: (i, 0)),
                  _whole((D, 3 * D)), _whole((D, 3 * D)),
                  _whole((1, 3 * D)), _whole((1, 3 * D)),
                  _whole((D, D)), _whole((D, D)), _whole((3, D)), _whole((1, D))],
        out_specs=[pl.BlockSpec((BN, D), lambda i: (i, 0))] * 3,
        out_shape=[fdt((N, D), jnp.float32)] * 3,
    )

    mesh = plsc.VectorSubcoreMesh(core_axis_name="c", subcore_axis_name="s")
    sc_params = pltpu.CompilerParams(use_tc_tiling_on_sc=False)
    PER_W = E2 // NW
    GCH = 800
    gather_call = pl.kernel(
        functools.partial(_gather_body, PER_W, GCH),
        out_type=(fdt((E2, D), jnp.float32), fdt((E2, D), jnp.float32)),
        mesh=mesh,
        compiler_params=sc_params,
        scratch_types=[pltpu.VMEM((GCH,), jnp.int32),
                       pltpu.VMEM((GCH,), jnp.int32),
                       pltpu.VMEM((GCH, D), jnp.float32),
                       pltpu.VMEM((GCH, D), jnp.float32),
                       pltpu.SemaphoreType.DMA],
    )

    scatter_call = pl.kernel(
        functools.partial(_scatter_body, HALF, E2),
        out_type=fdt((N, D), jnp.float32),
        mesh=mesh,
        compiler_params=sc_params,
        scratch_types=[pltpu.VMEM_SHARED((HALF + NS, D), jnp.float32),
                       pltpu.VMEM((8, 100), jnp.int32),
                       pltpu.VMEM((800, D), jnp.float32)],
    )

    nf, As, Ad = init_call(classes, pos, in_W, in_b2, ws, wd, wp, m1b2)
    for _ in range(ITERS):
        g0, g1 = gather_call(As, Ad, es0, es1)
        msg = mlp_call(g0, g1, m2_W, m2b2, m3_W, m3b2)
        a = scatter_call(msg, sidx, zstripe)
        nf, As, Ad = gru_call(a, nf, pos, gn_Wih, gn_Whh, bih2, bhh2,
                              ws, wd, wp, m1b2)
    return nf


# speedup vs baseline: 4.8754x; 1.0206x over previous
import functools

import jax
import jax.numpy as jnp
from jax import lax
from jax.experimental import pallas as pl
from jax.experimental.pallas import tpu as pltpu
from jax.experimental.pallas import tpu_sc as plsc

NC, NS = 2, 16
NW = NC * NS
D = 32
ITERS = 6


def _init_body(classes_ref, pos_ref, inw_ref, inb_ref, ws_ref, wd_ref, wp_ref,
               m1b_ref, nf_ref, as_ref, ad_ref):
    nf = jnp.dot(classes_ref[...], inw_ref[...],
                 preferred_element_type=jnp.float32) + inb_ref[...]
    posw = jnp.dot(pos_ref[...], wp_ref[...], preferred_element_type=jnp.float32)
    nf_ref[...] = nf
    as_ref[...] = jnp.dot(nf, ws_ref[...], preferred_element_type=jnp.float32) - posw
    ad_ref[...] = (jnp.dot(nf, wd_ref[...], preferred_element_type=jnp.float32)
                   + posw + m1b_ref[...])


def _mlp_body(g0_ref, g1_ref, w2_ref, b2_ref, w3_ref, b3_ref, msg_ref):
    h1 = jnp.maximum(g0_ref[...] + g1_ref[...], 0.0)
    h2 = jnp.maximum(
        jnp.dot(h1, w2_ref[...], preferred_element_type=jnp.float32) + b2_ref[...],
        0.0)
    msg_ref[...] = (jnp.dot(h2, w3_ref[...], preferred_element_type=jnp.float32)
                    + b3_ref[...])


def _gru_body(a_ref, nf_ref, pos_ref, wih_ref, whh_ref, bih_ref, bhh_ref,
              ws_ref, wd_ref, wp_ref, m1b_ref, nfo_ref, as_ref, ad_ref):
    a = a_ref[...]
    h = nf_ref[...]
    gi = jnp.dot(a, wih_ref[...], preferred_element_type=jnp.float32) + bih_ref[...]
    gh = jnp.dot(h, whh_ref[...], preferred_element_type=jnp.float32) + bhh_ref[...]
    r = jax.nn.sigmoid(gi[:, 0:D] + gh[:, 0:D])
    z = jax.nn.sigmoid(gi[:, D:2 * D] + gh[:, D:2 * D])
    n = jnp.tanh(gi[:, 2 * D:3 * D] + r * gh[:, 2 * D:3 * D])
    nf_new = (1.0 - z) * n + z * h
    posw = jnp.dot(pos_ref[...], wp_ref[...], preferred_element_type=jnp.float32)
    nfo_ref[...] = nf_new
    as_ref[...] = (jnp.dot(nf_new, ws_ref[...], preferred_element_type=jnp.float32)
                   - posw)
    ad_ref[...] = (jnp.dot(nf_new, wd_ref[...], preferred_element_type=jnp.float32)
                   + posw + m1b_ref[...])


def _whole(shape):
    return pl.BlockSpec(shape, lambda i: (0,) * len(shape))


def _gather_body(per_w, gch, as_hbm, ad_hbm, i0_hbm, i1_hbm, g0_hbm, g1_hbm,
                 idx0_v, idx1_v, r0_v, r1_v, sem):
    wid = lax.axis_index("s") * NC + lax.axis_index("c")
    base = wid * per_w

    def step(k, carry):
        off = base + k * gch
        d0 = pltpu.async_copy(i0_hbm.at[pl.ds(off, gch)], idx0_v, sem)
        d1 = pltpu.async_copy(i1_hbm.at[pl.ds(off, gch)], idx1_v, sem)
        d0.wait()
        d1.wait()
        q0 = pltpu.async_copy(as_hbm.at[idx0_v], r0_v, sem)
        q1 = pltpu.async_copy(ad_hbm.at[idx1_v], r1_v, sem)
        q0.wait()
        q1.wait()
        w0 = pltpu.async_copy(r0_v, g0_hbm.at[pl.ds(off, gch)], sem)
        w1 = pltpu.async_copy(r1_v, g1_hbm.at[pl.ds(off, gch)], sem)
        w0.wait()
        w1.wait()
        return carry

    lax.fori_loop(0, per_w // gch, step, 0)


def _scatter_body(half, e2, msg_hbm, sidx_hbm, z_hbm, out_hbm, acc_sh, idx_v,
                  m_v, lsem, ssem):
    c = lax.axis_index("c")
    s = lax.axis_index("s")
    sp_stripe = (half + NS) // NS
    out_stripe = half // NS
    n_idx_rows = e2 // 100
    rows_per_s = n_idx_rows // NS
    kr = 8
    n_chunks = rows_per_s // kr

    pltpu.sync_copy(z_hbm, acc_sh.at[pl.ds(s * sp_stripe, sp_stripe)])
    plsc.subcore_barrier()

    def step(k, carry):
        irow = c * n_idx_rows + s * rows_per_s + k * kr
        mrow = (s * rows_per_s + k * kr) * 100
        di = pltpu.async_copy(sidx_hbm.at[pl.ds(irow, kr)], idx_v, lsem)
        dm = pltpu.async_copy(msg_hbm.at[pl.ds(mrow, kr * 100)], m_v, lsem)
        di.wait()
        dm.wait()
        adds = [pltpu.async_copy(m_v.at[pl.ds(j * 100, 100)],
                                 acc_sh.at[idx_v.at[j]], ssem, add=True)
                for j in range(kr)]
        for d in adds:
            d.wait()
        return carry

    lax.fori_loop(0, n_chunks, step, 0)
    plsc.subcore_barrier()
    pltpu.sync_copy(acc_sh.at[pl.ds(s * out_stripe, out_stripe)],
                    out_hbm.at[pl.ds(c * half + s * out_stripe, out_stripe)])


def kernel(pos, classes, edges, in_W, in_b, m1_W, m1_b, m2_W, m2_b, m3_W, m3_b,
           gn_Wih, gn_Whh, gn_bih, gn_bhh, ge_Wih, ge_Whh, ge_bih, ge_bhh):
    N = pos.shape[0]
    E = edges.shape[1]
    E2 = 2 * E
    HALF = N // 2
    C = classes.shape[1]

    es0 = jnp.concatenate([edges[0], edges[1]])
    es1 = jnp.concatenate([edges[1], edges[0]])
    i_lo = jnp.where(es0 < HALF, es0, HALF)
    i_hi = jnp.where(es0 >= HALF, es0 - HALF, HALF)
    sidx = jnp.concatenate([i_lo, i_hi]).reshape(2 * (E2 // 100), 100)
    ws = m1_W[0:D]
    wd = m1_W[D:2 * D]
    wp = m1_W[2 * D:]
    in_b2 = in_b.reshape(1, D)
    m1b2 = m1_b.reshape(1, D)
    m2b2 = m2_b.reshape(1, D)
    m3b2 = m3_b.reshape(1, D)
    bih2 = gn_bih.reshape(1, 3 * D)
    bhh2 = gn_bhh.reshape(1, 3 * D)
    sp_stripe = (HALF + NS) // NS
    zstripe = jnp.zeros((sp_stripe, D), jnp.float32)

    BN = 2000
    fdt = jax.ShapeDtypeStruct
    init_call = pl.pallas_call(
        _init_body,
        grid=(N // BN,),
        in_specs=[pl.BlockSpec((BN, C), lambda i: (i, 0)),
                  pl.BlockSpec((BN, 3), lambda i: (i, 0)),
                  _whole((C, D)), _whole((1, D)), _whole((D, D)),
                  _whole((D, D)), _whole((3, D)), _whole((1, D))],
        out_specs=[pl.BlockSpec((BN, D), lambda i: (i, 0))] * 3,
        out_shape=[fdt((N, D), jnp.float32)] * 3,
    )

    BE = 8000
    mlp_call = pl.pallas_call(
        _mlp_body,
        grid=(E2 // BE,),
        in_specs=[pl.BlockSpec((BE, D), lambda i: (i, 0)),
                  pl.BlockSpec((BE, D), lambda i: (i, 0)),
                  _whole((D, D)), _whole((1, D)), _whole((D, D)), _whole((1, D))],
        out_specs=pl.BlockSpec((BE, D), lambda i: (i, 0)),
        out_shape=fdt((E2, D), jnp.float32),
    )

    gru_call = pl.pallas_call(
        _gru_body,
        grid=(N // BN,),
        in_specs=[pl.BlockSpec((BN, D), lambda i: (i, 0)),
                  pl.BlockSpec((BN, D), lambda i: (i, 0)),
                  pl.BlockSpec((BN, 3), lambda i: (i, 0)),
                  _whole((D, 3 * D)), _whole((D, 3 * D)),
                  _whole((1, 3 * D)), _whole((1, 3 * D)),
                  _whole((D, D)), _whole((D, D)), _whole((3, D)), _whole((1, D))],
        out_specs=[pl.BlockSpec((BN, D), lambda i: (i, 0))] * 3,
        out_shape=[fdt((N, D), jnp.float32)] * 3,
    )

    mesh = plsc.VectorSubcoreMesh(core_axis_name="c", subcore_axis_name="s")
    sc_params = pltpu.CompilerParams(use_tc_tiling_on_sc=False)
    PER_W = E2 // NW
    GCH = 800
    gather_call = pl.kernel(
        functools.partial(_gather_body, PER_W, GCH),
        out_type=(fdt((E2, D), jnp.float32), fdt((E2, D), jnp.float32)),
        mesh=mesh,
        compiler_params=sc_params,
        scratch_types=[pltpu.VMEM((GCH,), jnp.int32),
                       pltpu.VMEM((GCH,), jnp.int32),
                       pltpu.VMEM((GCH, D), jnp.float32),
                       pltpu.VMEM((GCH, D), jnp.float32),
                       pltpu.SemaphoreType.DMA],
    )

    scatter_call = pl.kernel(
        functools.partial(_scatter_body, HALF, E2),
        out_type=fdt((N, D), jnp.float32),
        mesh=mesh,
        compiler_params=sc_params,
        scratch_types=[pltpu.VMEM_SHARED((HALF + NS, D), jnp.float32),
                       pltpu.VMEM((8, 100), jnp.int32),
                       pltpu.VMEM((800, D), jnp.float32),
                       pltpu.SemaphoreType.DMA,
                       pltpu.SemaphoreType.DMA],
    )

    nf, As, Ad = init_call(classes, pos, in_W, in_b2, ws, wd, wp, m1b2)
    for _ in range(ITERS):
        g0, g1 = gather_call(As, Ad, es0, es1)
        msg = mlp_call(g0, g1, m2_W, m2b2, m3_W, m3b2)
        a = scatter_call(msg, sidx, zstripe)
        nf, As, Ad = gru_call(a, nf, pos, gn_Wih, gn_Whh, bih2, bhh2,
                              ws, wd, wp, m1b2)
    return nf
